# Initial kernel scaffold; baseline (speedup 1.0000x reference)
import jax
import jax.numpy as jnp
from jax.experimental import pallas as pl


def kernel(x, memory_bank, Wq_s, Wk_s, Wv_s, Wo_s, Wq_c, Wk_c, Wv_c, Wo_c, ln1_g, ln1_b, ln2_g, ln2_b, ln3_g, ln3_b, Wf1, bf1, Wf2, bf2, W_mu, b_mu, W_ls, b_ls, We1, be1, We2, be2):
    raise NotImplementedError("write your pallas kernel here")


# speedup vs baseline: 2.2511x; 2.2511x over previous
import functools
import math

import jax
import jax.numpy as jnp
from jax.experimental import pallas as pl
from jax.experimental.pallas import tpu as pltpu

_B = 2
_T = 2048
_D = 1024
_HID = 4096
_E = 8
_TOPK = 2
_L = 2
_H = 4
_HIST = 10
_FF = 4 * _D
_MEM = 512

_BT = _B * _T
_P = _TOPK * _BT
_BLK = 256
_NP = _P + _E * _BLK
_NB = _NP // _BLK
_TB = 512
_C = 0.5 * math.log(2.0 * math.pi)


def _layernorm(x, g, b):
    m = jnp.mean(x, axis=-1, keepdims=True)
    v = jnp.var(x, axis=-1, keepdims=True)
    return (x - m) / jnp.sqrt(v + 1e-5) * g + b


def _mha(q_in, kv_in, Wq, Wk, Wv, Wo):
    Bq, Tq, d = q_in.shape
    Tk = kv_in.shape[1]
    hd = d // _H
    q = (q_in @ Wq).reshape(Bq, Tq, _H, hd).transpose(0, 2, 1, 3)
    k = (kv_in @ Wk).reshape(Bq, Tk, _H, hd).transpose(0, 2, 1, 3)
    v = (kv_in @ Wv).reshape(Bq, Tk, _H, hd).transpose(0, 2, 1, 3)
    att = jax.nn.softmax(q @ k.transpose(0, 1, 3, 2) / math.sqrt(hd), axis=-1)
    o = (att @ v).transpose(0, 2, 1, 3).reshape(Bq, Tq, d)
    return o @ Wo


def _route_body(x_ref, mu_ref, sig_ref, ls_ref, lp_ref, ti_ref, w_ref):
    xb = x_ref[0]
    lps = []
    for e in range(_E):
        diff = (xb - mu_ref[0, e]) / sig_ref[0, e]
        terms = -0.5 * diff * diff - ls_ref[0, e] - _C
        lps.append(jnp.sum(terms, axis=-1))
    lp = jnp.stack(lps, axis=-1)
    lp_ref[0] = lp
    iota = jax.lax.broadcasted_iota(jnp.int32, (_TB, _E), 1)
    m1 = jnp.max(lp, axis=1, keepdims=True)
    i1 = jnp.min(jnp.where(lp == m1, iota, _E), axis=1)
    masked = jnp.where(iota == i1[:, None], -jnp.inf, lp)
    m2 = jnp.max(masked, axis=1, keepdims=True)
    i2 = jnp.min(jnp.where(masked == m2, iota, _E), axis=1)
    ti_ref[0] = jnp.stack([i1, i2], axis=-1)
    e2 = jnp.exp(m2[:, 0] - m1[:, 0])
    denom = 1.0 + e2
    w_ref[0] = jnp.stack([1.0 / denom, e2 / denom], axis=-1)


def _route(x, mus, sig, ls):
    grid = (_B, _T // _TB)
    return pl.pallas_call(
        _route_body,
        grid=grid,
        in_specs=[
            pl.BlockSpec((1, _TB, _D), lambda b, i: (b, i, 0)),
            pl.BlockSpec((1, _E, _D), lambda b, i: (b, 0, 0)),
            pl.BlockSpec((1, _E, _D), lambda b, i: (b, 0, 0)),
            pl.BlockSpec((1, _E, _D), lambda b, i: (b, 0, 0)),
        ],
        out_specs=[
            pl.BlockSpec((1, _TB, _E), lambda b, i: (b, i, 0)),
            pl.BlockSpec((1, _TB, _TOPK), lambda b, i: (b, i, 0)),
            pl.BlockSpec((1, _TB, _TOPK), lambda b, i: (b, i, 0)),
        ],
        out_shape=[
            jax.ShapeDtypeStruct((_B, _T, _E), jnp.float32),
            jax.ShapeDtypeStruct((_B, _T, _TOPK), jnp.int32),
            jax.ShapeDtypeStruct((_B, _T, _TOPK), jnp.float32),
        ],
    )(x, mus, sig, ls)


_HB = 1024
_NH = _HID // _HB


def _mlp_body(be_ref, xs_ref, gw_ref, w1_ref, b1_ref, w2_ref, b2_ref, out_ref):
    j = pl.program_id(1)
    h = jnp.dot(xs_ref[...], w1_ref[0], preferred_element_type=jnp.float32)
    h = h + b1_ref[0]
    h = 0.5 * h * (1.0 + jax.lax.erf(h * (1.0 / math.sqrt(2.0))))
    acc = jnp.dot(h, w2_ref[0], preferred_element_type=jnp.float32)

    @pl.when(j == 0)
    def _():
        out_ref[...] = acc

    @pl.when(j > 0)
    def _():
        out_ref[...] += acc

    @pl.when(j == _NH - 1)
    def _():
        out_ref[...] = gw_ref[...] * (out_ref[...] + b2_ref[0])


def _expert_mlp(xs, gw, We1, be1, We2, be2, block_expert):
    grid_spec = pltpu.PrefetchScalarGridSpec(
        num_scalar_prefetch=1,
        grid=(_NB, _NH),
        in_specs=[
            pl.BlockSpec((_BLK, _D), lambda i, j, be: (i, 0)),
            pl.BlockSpec((_BLK, 1), lambda i, j, be: (i, 0)),
            pl.BlockSpec((1, _D, _HB), lambda i, j, be: (be[i], 0, j)),
            pl.BlockSpec((1, 1, _HB), lambda i, j, be: (be[i], 0, j)),
            pl.BlockSpec((1, _HB, _D), lambda i, j, be: (be[i], j, 0)),
            pl.BlockSpec((1, 1, _D), lambda i, j, be: (be[i], 0, 0)),
        ],
        out_specs=pl.BlockSpec((_BLK, _D), lambda i, j, be: (i, 0)),
    )
    return pl.pallas_call(
        _mlp_body,
        grid_spec=grid_spec,
        out_shape=jax.ShapeDtypeStruct((_NP, _D), jnp.float32),
    )(block_expert, xs, gw, We1, be1.reshape(_E, 1, _HID), We2,
      be2.reshape(_E, 1, _D))


def kernel(x, memory_bank, Wq_s, Wk_s, Wv_s, Wo_s, Wq_c, Wk_c, Wv_c, Wo_c,
           ln1_g, ln1_b, ln2_g, ln2_b, ln3_g, ln3_b, Wf1, bf1, Wf2, bf2,
           W_mu, b_mu, W_ls, b_ls, We1, be1, We2, be2):
    q_subject = jnp.mean(x, axis=1)
    sim = q_subject @ memory_bank.T
    _, idx = jax.lax.top_k(sim, _HIST)
    retrieved = memory_bank[idx]
    tgt = q_subject[:, None, :]
    for l in range(_L):
        h1 = _layernorm(tgt, ln1_g[l], ln1_b[l])
        tgt = tgt + _mha(h1, h1, Wq_s[l], Wk_s[l], Wv_s[l], Wo_s[l])
        h2 = _layernorm(tgt, ln2_g[l], ln2_b[l])
        tgt = tgt + _mha(h2, retrieved, Wq_c[l], Wk_c[l], Wv_c[l], Wo_c[l])
        h3 = _layernorm(tgt, ln3_g[l], ln3_b[l])
        tgt = tgt + jax.nn.gelu(h3 @ Wf1[l] + bf1[l], approximate=False) @ Wf2[l] + bf2[l]
    mus = (tgt @ W_mu + b_mu).reshape(_B, _E, _D)
    ls = (tgt @ W_ls + b_ls).reshape(_B, _E, _D)
    sig = jnp.exp(ls)

    lp, ti, w = _route(x, mus, sig, ls)

    ek = ti.reshape(_BT, _TOPK).T.reshape(_P)
    wk = w.reshape(_BT, _TOPK).T.reshape(_P)
    perm = jnp.argsort(ek, stable=True)
    e_sorted = ek[perm]
    counts = jnp.bincount(ek, length=_E)
    starts = jnp.concatenate([jnp.zeros(1, jnp.int32),
                              jnp.cumsum(counts)[:-1].astype(jnp.int32)])
    padded = ((counts + _BLK - 1) // _BLK) * _BLK
    pstart = jnp.concatenate([jnp.zeros(1, jnp.int32),
                              jnp.cumsum(padded).astype(jnp.int32)])
    rank = jnp.arange(_P, dtype=jnp.int32) - starts[e_sorted]
    pos = pstart[e_sorted] + rank
    tok = (perm % _BT).astype(jnp.int32)
    gidx = jnp.zeros(_NP, jnp.int32).at[pos].set(tok)
    gw = jnp.zeros(_NP, jnp.float32).at[pos].set(wk[perm])
    inv = jnp.zeros(_P, jnp.int32).at[perm].set(pos)
    rb = jnp.arange(_NB, dtype=jnp.int32) * _BLK
    block_expert = jnp.clip(
        jnp.searchsorted(pstart[1:], rb, side='right'), 0, _E - 1
    ).astype(jnp.int32)

    x_flat = x.reshape(_BT, _D)
    xs = x_flat[gidx]

    out_sorted = _expert_mlp(xs, gw.reshape(_NP, 1), We1, be1, We2, be2,
                             block_expert)

    y = out_sorted[inv[:_BT]] + out_sorted[inv[_BT:]]
    final = y.reshape(_B, _T, _D)
    return final, lp, ti
